# baseline (speedup 1.0000x reference)
import functools

import jax
import jax.numpy as jnp
from jax import lax
from jax.experimental import pallas as pl
from jax.experimental.pallas import tpu as pltpu
from jax.experimental.pallas import tpu_sc as plsc

NC = 2
NS = 16
NW = NC * NS

CHUNK = 128


@functools.partial(jax.jit, static_argnums=(2, 3))
def _sc_gather(idx_flat, weight, n_rows, d):
    b_per_w = n_rows // NW
    n_chunks = b_per_w // CHUNK
    mesh = plsc.VectorSubcoreMesh(
        core_axis_name="c", subcore_axis_name="s", num_cores=NC, num_subcores=NS
    )

    @functools.partial(
        pl.kernel,
        out_type=jax.ShapeDtypeStruct((n_rows, d), jnp.float32),
        mesh=mesh,
        compiler_params=pltpu.CompilerParams(use_tc_tiling_on_sc=False),
        scratch_types=[
            pltpu.VMEM((CHUNK,), jnp.int32),
            pltpu.VMEM((CHUNK, d), jnp.float32),
            pltpu.SemaphoreType.DMA,
        ],
    )
    def k(idx_hbm, table_hbm, out_hbm, idx_v, rows_v, sem):
        wid = lax.axis_index("s") * NC + lax.axis_index("c")
        base = wid * b_per_w

        def step(i, carry):
            off = pl.multiple_of(base + i * CHUNK, CHUNK)
            pltpu.sync_copy(idx_hbm.at[pl.ds(off, CHUNK)], idx_v)
            pltpu.async_copy(table_hbm.at[idx_v], rows_v, sem).wait()
            pltpu.sync_copy(rows_v, out_hbm.at[pl.ds(off, CHUNK)])
            return carry

        lax.fori_loop(0, n_chunks, step, 0)

    return k(idx_flat, weight)


def kernel(inputs, weight):
    b, s = inputs.shape
    v, d = weight.shape
    idx_flat = inputs.reshape(-1).astype(jnp.int32)
    out = _sc_gather(idx_flat, weight, b * s, d)
    return out.reshape(b, s, d)


# speedup vs baseline: 1.1910x; 1.1910x over previous
import functools

import jax
import jax.numpy as jnp
from jax import lax
from jax.experimental import pallas as pl
from jax.experimental.pallas import tpu as pltpu
from jax.experimental.pallas import tpu_sc as plsc

NC = 2
NS = 16
NW = NC * NS

CHUNK = 128
K = 8


@functools.partial(jax.jit, static_argnums=(2, 3))
def _sc_gather(idx3, weight, n_chunks, d):
    n_grp = n_chunks // K
    mesh = plsc.VectorSubcoreMesh(
        core_axis_name="c", subcore_axis_name="s", num_cores=NC, num_subcores=NS
    )

    @functools.partial(
        pl.kernel,
        out_type=jax.ShapeDtypeStruct((NW, n_chunks, CHUNK, d), jnp.float32),
        mesh=mesh,
        compiler_params=pltpu.CompilerParams(use_tc_tiling_on_sc=False),
        scratch_types=[
            pltpu.VMEM((n_chunks, CHUNK), jnp.int32),
            pltpu.VMEM((K, CHUNK, d), jnp.float32),
            pltpu.SemaphoreType.DMA((K,)),
        ],
    )
    def k(idx_hbm, table_hbm, out_hbm, idx_v, bufs, gsem):
        wid = lax.axis_index("s") * NC + lax.axis_index("c")
        pltpu.sync_copy(idx_hbm.at[wid], idx_v)
        for b in range(K):
            pltpu.async_copy(table_hbm.at[idx_v.at[b]], bufs.at[b], gsem.at[b])

        def grp(g, carry):
            for b in range(K):
                j = g * K + b
                pltpu.make_async_copy(
                    table_hbm.at[idx_v.at[b]], bufs.at[b], gsem.at[b]
                ).wait()
                pltpu.sync_copy(bufs.at[b], out_hbm.at[wid, j])
                jn = jnp.minimum(j + K, n_chunks - 1)
                pltpu.async_copy(table_hbm.at[idx_v.at[jn]], bufs.at[b], gsem.at[b])
            return carry

        lax.fori_loop(0, n_grp, grp, 0)
        for b in range(K):
            pltpu.make_async_copy(
                table_hbm.at[idx_v.at[b]], bufs.at[b], gsem.at[b]
            ).wait()

    return k(idx3, weight)


def kernel(inputs, weight):
    b, s = inputs.shape
    v, d = weight.shape
    n_rows = b * s
    n_chunks = n_rows // (NW * CHUNK)
    idx3 = inputs.reshape(NW, n_chunks, CHUNK).astype(jnp.int32)
    out = _sc_gather(idx3, weight, n_chunks, d)
    return out.reshape(b, s, d)
